# Initial kernel scaffold; baseline (speedup 1.0000x reference)
import jax
import jax.numpy as jnp

N = 50000
E = 1600000
T = 10


def kernel(params, edge_index):
    p0, beta, gamma = params[0], params[1], params[2]
    base = jax.random.key(12345)

    def gdiff(idx):
        u = jax.random.uniform(jax.random.fold_in(base, idx), (N, 2),
                               minval=1e-6, maxval=1.0 - 1e-6)
        g = -jnp.log(-jnp.log(u))
        return g[:, 0] - g[:, 1]

    c0 = jnp.log(1.0 - p0) - jnp.log(p0)
    log1mb = jnp.log(1.0 - beta)
    c_rec = jnp.log(1.0 - gamma) - jnp.log(gamma)

    src, dst = edge_index[0], edge_index[1]

    inf = (gdiff(0) > c0).astype(jnp.float32)
    sus = 1.0 - inf
    hist_i = [inf.sum()]
    rec_total = jnp.float32(0.0)
    hist_r = [rec_total]
    for t in range(T):
        cnt = jax.ops.segment_sum(inf[src], dst, num_segments=N)
        nm = cnt * sus
        a = nm * log1mb
        thr = a - jnp.log(1.0 - jnp.exp(a))
        new_i = (gdiff(3 * t + 1) > thr).astype(jnp.float32)
        new_r = inf * (gdiff(3 * t + 2) > c_rec).astype(jnp.float32)
        inf = inf + new_i - new_r
        sus = sus - new_i
        rec_total = rec_total + new_r.sum()
        hist_i.append(inf.sum())
        hist_r.append(rec_total)
    return jnp.stack(hist_i), jnp.stack(hist_r)


# speedup vs baseline: 224.9723x; 224.9723x over previous
import functools

import jax
import jax.numpy as jnp
from jax import lax
from jax.experimental import pallas as pl
from jax.experimental.pallas import tpu as pltpu
from jax.experimental.pallas import tpu_sc as plsc

N = 50000
E = 1600000
T = 10

LANES = 128
ROWS = 392
NPAD = ROWS * LANES
NW = 32
EPW = E // NW
CH = 2000
NCH = EPW // CH
VPC = CH // 16


def _sc_count_body(inf_hbm, src_hbm, dst_hbm, out_hbm, inf_v, cnt_v, src_v,
                   dst_v):
    cid = lax.axis_index("c")
    sid = lax.axis_index("s")
    wid = sid * 2 + cid

    pltpu.sync_copy(inf_hbm, inf_v)

    def zero_body(i, carry):
        cnt_v[pl.ds(i * 16, 16)] = jnp.zeros((16,), jnp.float32)
        return carry

    lax.fori_loop(0, NPAD // 16, zero_body, 0)

    base = wid * EPW

    def chunk_body(ci, carry):
        off = base + ci * CH
        pltpu.sync_copy(src_hbm.at[pl.ds(off, CH)], src_v)
        pltpu.sync_copy(dst_hbm.at[pl.ds(off, CH)], dst_v)

        def vec_body(i, carry2):
            sidx = src_v[pl.ds(i * 16, 16)]
            didx = dst_v[pl.ds(i * 16, 16)]
            vals = plsc.load_gather(inf_v, [sidx])
            plsc.addupdate_scatter(cnt_v, [didx], vals)
            return carry2

        lax.fori_loop(0, VPC, vec_body, 0)
        return carry

    lax.fori_loop(0, NCH, chunk_body, 0)
    pltpu.sync_copy(cnt_v, out_hbm.at[wid])


_sc_count = functools.partial(
    pl.kernel,
    mesh=plsc.VectorSubcoreMesh(core_axis_name="c", subcore_axis_name="s"),
    compiler_params=pltpu.CompilerParams(
        needs_layout_passes=False, use_tc_tiling_on_sc=False),
    out_type=jax.ShapeDtypeStruct((NW, NPAD), jnp.float32),
    scratch_types=[
        pltpu.VMEM((NPAD,), jnp.float32),
        pltpu.VMEM((NPAD,), jnp.float32),
        pltpu.VMEM((CH,), jnp.int32),
        pltpu.VMEM((CH,), jnp.int32),
    ],
)(_sc_count_body)


def _init_body(d0_ref, scal_ref, inf_ref, sus_ref, sums_ref):
    c0 = scal_ref[0]
    r = lax.broadcasted_iota(jnp.int32, (ROWS, LANES), 0)
    c = lax.broadcasted_iota(jnp.int32, (ROWS, LANES), 1)
    valid = (r * LANES + c < N).astype(jnp.float32)
    inf0 = valid * (d0_ref[...] > c0).astype(jnp.float32)
    inf_ref[...] = inf0
    sus_ref[...] = valid - inf0
    si = jnp.sum(inf0)
    rr = lax.broadcasted_iota(jnp.int32, (8, LANES), 0)
    cc = lax.broadcasted_iota(jnp.int32, (8, LANES), 1)
    sums_ref[...] = jnp.where((rr == 0) & (cc == 0), si, 0.0)


_init_call = pl.pallas_call(
    _init_body,
    out_shape=(
        jax.ShapeDtypeStruct((ROWS, LANES), jnp.float32),
        jax.ShapeDtypeStruct((ROWS, LANES), jnp.float32),
        jax.ShapeDtypeStruct((8, LANES), jnp.float32),
    ),
    in_specs=[
        pl.BlockSpec(memory_space=pltpu.VMEM),
        pl.BlockSpec(memory_space=pltpu.SMEM),
    ],
)


def _update_body(part_ref, inf_ref, sus_ref, dinf_ref, drec_ref, scal_ref,
                 ninf_ref, nsus_ref, sums_ref):
    log1mb = scal_ref[1]
    c_rec = scal_ref[2]
    cnt = jnp.sum(part_ref[...], axis=0)
    inf = inf_ref[...]
    sus = sus_ref[...]
    a = cnt * sus * log1mb
    thr = a - jnp.log(1.0 - jnp.exp(a))
    new_i = (dinf_ref[...] > thr).astype(jnp.float32)
    new_r = inf * (drec_ref[...] > c_rec).astype(jnp.float32)
    ninf = inf + new_i - new_r
    ninf_ref[...] = ninf
    nsus_ref[...] = sus - new_i
    si = jnp.sum(ninf)
    sr = jnp.sum(new_r)
    rr = lax.broadcasted_iota(jnp.int32, (8, LANES), 0)
    cc = lax.broadcasted_iota(jnp.int32, (8, LANES), 1)
    sums_ref[...] = jnp.where(
        (rr == 0) & (cc == 0), si,
        jnp.where((rr == 0) & (cc == 1), sr, 0.0))


_update_call = pl.pallas_call(
    _update_body,
    out_shape=(
        jax.ShapeDtypeStruct((ROWS, LANES), jnp.float32),
        jax.ShapeDtypeStruct((ROWS, LANES), jnp.float32),
        jax.ShapeDtypeStruct((8, LANES), jnp.float32),
    ),
    in_specs=[
        pl.BlockSpec(memory_space=pltpu.VMEM),
        pl.BlockSpec(memory_space=pltpu.VMEM),
        pl.BlockSpec(memory_space=pltpu.VMEM),
        pl.BlockSpec(memory_space=pltpu.VMEM),
        pl.BlockSpec(memory_space=pltpu.VMEM),
        pl.BlockSpec(memory_space=pltpu.SMEM),
    ],
)


def kernel(params, edge_index):
    dtype = params.dtype
    p0, beta, gamma = params[0], params[1], params[2]
    base = jax.random.key(12345)

    def gdiff(idx):
        u = jax.random.uniform(jax.random.fold_in(base, idx), (N, 2),
                               minval=1e-6, maxval=1.0 - 1e-6)
        g = -jnp.log(-jnp.log(u))
        d = g[:, 0] - g[:, 1]
        return jnp.pad(d, (0, NPAD - N)).reshape(ROWS, LANES)

    d0 = gdiff(0)
    dinf = [gdiff(3 * t + 1) for t in range(T)]
    drec = [gdiff(3 * t + 2) for t in range(T)]

    c0 = jnp.log(1.0 - p0) - jnp.log(p0)
    log1mb = jnp.log(1.0 - beta)
    c_rec = jnp.log(1.0 - gamma) - jnp.log(gamma)
    scal = jnp.stack([c0, log1mb, c_rec, jnp.float32(0.0)])

    src = edge_index[0].astype(jnp.int32)
    dst = edge_index[1].astype(jnp.int32)

    inf, sus, sums = _init_call(d0, scal)
    hist_i = [sums[0, 0]]
    rec_d = []
    for t in range(T):
        partials = _sc_count(inf.reshape(NPAD), src, dst)
        partials = partials.reshape(NW, ROWS, LANES)
        inf, sus, sums = _update_call(partials, inf, sus, dinf[t], drec[t],
                                      scal)
        hist_i.append(sums[0, 0])
        rec_d.append(sums[0, 1])

    infected_hist = jnp.stack(hist_i).astype(dtype)
    recovered_hist = jnp.concatenate(
        [jnp.zeros((1,), dtype), jnp.cumsum(jnp.stack(rec_d))]).astype(dtype)
    return (infected_hist, recovered_hist)


# speedup vs baseline: 418.1630x; 1.8587x over previous
import functools

import jax
import jax.numpy as jnp
from jax import lax
from jax.experimental import pallas as pl
from jax.experimental.pallas import tpu as pltpu
from jax.experimental.pallas import tpu_sc as plsc

N = 50000
E = 1600000
T = 10

LANES = 128
ROWS = 392
NPAD = ROWS * LANES
NW = 32
EPW = E // NW
CH = 10000
NCH = EPW // CH


def _sc_count_body(inf_hbm, src_hbm, dst_hbm, out_hbm, inf_v, cnt_v, src_v,
                   dst_v, sem):
    cid = lax.axis_index("c")
    sid = lax.axis_index("s")
    wid = sid * 2 + cid

    stage = pltpu.async_copy(inf_hbm, inf_v, sem)

    @plsc.parallel_loop(0, NPAD, 16, unroll=8)
    def _zero(i):
        cnt_v[pl.ds(i, 16)] = jnp.zeros((16,), jnp.float32)

    stage.wait()

    base = wid * EPW

    def chunk_body(ci, carry):
        off = base + ci * CH
        pltpu.sync_copy(src_hbm.at[pl.ds(off, CH)], src_v)
        pltpu.sync_copy(dst_hbm.at[pl.ds(off, CH)], dst_v)

        @plsc.parallel_loop(0, CH, 16, unroll=5)
        def _edges(i):
            sidx = src_v[pl.ds(i, 16)]
            didx = dst_v[pl.ds(i, 16)]
            vals = plsc.load_gather(inf_v, [sidx])
            plsc.addupdate_scatter(cnt_v, [didx], vals)

        return carry

    lax.fori_loop(0, NCH, chunk_body, 0)
    pltpu.sync_copy(cnt_v, out_hbm.at[wid])


_sc_count = functools.partial(
    pl.kernel,
    mesh=plsc.VectorSubcoreMesh(core_axis_name="c", subcore_axis_name="s"),
    compiler_params=pltpu.CompilerParams(
        needs_layout_passes=False, use_tc_tiling_on_sc=False),
    out_type=jax.ShapeDtypeStruct((NW, NPAD), jnp.float32),
    scratch_types=[
        pltpu.VMEM((NPAD,), jnp.float32),
        pltpu.VMEM((NPAD,), jnp.float32),
        pltpu.VMEM((CH,), jnp.int32),
        pltpu.VMEM((CH,), jnp.int32),
        pltpu.SemaphoreType.DMA,
    ],
)(_sc_count_body)


def _init_body(d0_ref, scal_ref, inf_ref, sus_ref, sums_ref):
    c0 = scal_ref[0]
    r = lax.broadcasted_iota(jnp.int32, (ROWS, LANES), 0)
    c = lax.broadcasted_iota(jnp.int32, (ROWS, LANES), 1)
    valid = (r * LANES + c < N).astype(jnp.float32)
    inf0 = valid * (d0_ref[...] > c0).astype(jnp.float32)
    inf_ref[...] = inf0
    sus_ref[...] = valid - inf0
    si = jnp.sum(inf0)
    rr = lax.broadcasted_iota(jnp.int32, (8, LANES), 0)
    cc = lax.broadcasted_iota(jnp.int32, (8, LANES), 1)
    sums_ref[...] = jnp.where((rr == 0) & (cc == 0), si, 0.0)


_init_call = pl.pallas_call(
    _init_body,
    out_shape=(
        jax.ShapeDtypeStruct((ROWS, LANES), jnp.float32),
        jax.ShapeDtypeStruct((ROWS, LANES), jnp.float32),
        jax.ShapeDtypeStruct((8, LANES), jnp.float32),
    ),
    in_specs=[
        pl.BlockSpec(memory_space=pltpu.VMEM),
        pl.BlockSpec(memory_space=pltpu.SMEM),
    ],
)


def _update_body(part_ref, inf_ref, sus_ref, dinf_ref, drec_ref, scal_ref,
                 ninf_ref, nsus_ref, sums_ref):
    log1mb = scal_ref[1]
    c_rec = scal_ref[2]
    cnt = jnp.sum(part_ref[...], axis=0)
    inf = inf_ref[...]
    sus = sus_ref[...]
    a = cnt * sus * log1mb
    thr = a - jnp.log(1.0 - jnp.exp(a))
    new_i = (dinf_ref[...] > thr).astype(jnp.float32)
    new_r = inf * (drec_ref[...] > c_rec).astype(jnp.float32)
    ninf = inf + new_i - new_r
    ninf_ref[...] = ninf
    nsus_ref[...] = sus - new_i
    si = jnp.sum(ninf)
    sr = jnp.sum(new_r)
    rr = lax.broadcasted_iota(jnp.int32, (8, LANES), 0)
    cc = lax.broadcasted_iota(jnp.int32, (8, LANES), 1)
    sums_ref[...] = jnp.where(
        (rr == 0) & (cc == 0), si,
        jnp.where((rr == 0) & (cc == 1), sr, 0.0))


_update_call = pl.pallas_call(
    _update_body,
    out_shape=(
        jax.ShapeDtypeStruct((ROWS, LANES), jnp.float32),
        jax.ShapeDtypeStruct((ROWS, LANES), jnp.float32),
        jax.ShapeDtypeStruct((8, LANES), jnp.float32),
    ),
    in_specs=[
        pl.BlockSpec(memory_space=pltpu.VMEM),
        pl.BlockSpec(memory_space=pltpu.VMEM),
        pl.BlockSpec(memory_space=pltpu.VMEM),
        pl.BlockSpec(memory_space=pltpu.VMEM),
        pl.BlockSpec(memory_space=pltpu.VMEM),
        pl.BlockSpec(memory_space=pltpu.SMEM),
    ],
)


def kernel(params, edge_index):
    dtype = params.dtype
    p0, beta, gamma = params[0], params[1], params[2]
    base = jax.random.key(12345)

    def gdiff(idx):
        u = jax.random.uniform(jax.random.fold_in(base, idx), (N, 2),
                               minval=1e-6, maxval=1.0 - 1e-6)
        g = -jnp.log(-jnp.log(u))
        d = g[:, 0] - g[:, 1]
        return jnp.pad(d, (0, NPAD - N)).reshape(ROWS, LANES)

    d0 = gdiff(0)
    dinf = [gdiff(3 * t + 1) for t in range(T)]
    drec = [gdiff(3 * t + 2) for t in range(T)]

    c0 = jnp.log(1.0 - p0) - jnp.log(p0)
    log1mb = jnp.log(1.0 - beta)
    c_rec = jnp.log(1.0 - gamma) - jnp.log(gamma)
    scal = jnp.stack([c0, log1mb, c_rec, jnp.float32(0.0)])

    src = edge_index[0].astype(jnp.int32)
    dst = edge_index[1].astype(jnp.int32)

    inf, sus, sums = _init_call(d0, scal)
    hist_i = [sums[0, 0]]
    rec_d = []
    for t in range(T):
        partials = _sc_count(inf.reshape(NPAD), src, dst)
        partials = partials.reshape(NW, ROWS, LANES)
        inf, sus, sums = _update_call(partials, inf, sus, dinf[t], drec[t],
                                      scal)
        hist_i.append(sums[0, 0])
        rec_d.append(sums[0, 1])

    infected_hist = jnp.stack(hist_i).astype(dtype)
    recovered_hist = jnp.concatenate(
        [jnp.zeros((1,), dtype), jnp.cumsum(jnp.stack(rec_d))]).astype(dtype)
    return (infected_hist, recovered_hist)
